# baseline (speedup 1.0000x reference)
import functools

import jax
import jax.numpy as jnp
from jax import lax
from jax.experimental import pallas as pl
from jax.experimental.pallas import tpu as pltpu
from jax.experimental.pallas import tpu_sc as plsc

B, T = 8, 2048
D_MODEL = 1024
N_L1 = 8
N_L2 = 1024
BD = 256
TB = 512
NT = T // TB

NW = 32
RPW = (B * T) // NW
CHUNK = 128
NCHUNK = RPW // CHUNK
WPB = T // RPW

TB3 = 1024


def _ln(x, g, b, eps=1e-5):
    m = jnp.mean(x, axis=-1, keepdims=True)
    v = jnp.mean((x - m) ** 2, axis=-1, keepdims=True)
    return (x - m) / jnp.sqrt(v + eps) * g + b


def _stage1_body(idx_ref, temp_ref, x_ref, cb_ref, W1_ref, b1_ref, g1_ref,
                 bt1_ref, hard_ref, soft_ref, gidx_ref, cbn_ref):
    x = x_ref[0]
    temp = temp_ref[0]

    @pl.when(pl.program_id(1) == 0)
    def _():
        cb = cb_ref[0]
        cb_inv = 1.0 / jnp.maximum(
            jnp.sqrt(jnp.sum(cb * cb, axis=-1, keepdims=True)), 1e-12)
        cbn_ref[...] = (cb * cb_inv).T

    h0 = jnp.dot(x, W1_ref[...], preferred_element_type=jnp.float32) + b1_ref[...]
    h = _ln(h0, g1_ref[...], bt1_ref[...])
    hn = h / jnp.maximum(jnp.sqrt(jnp.sum(h * h, axis=-1, keepdims=True)), 1e-12)

    logits = jnp.dot(hn, cbn_ref[...], preferred_element_type=jnp.float32) / temp

    rowmax = jnp.max(logits, axis=-1, keepdims=True)
    e = jnp.exp(logits - rowmax)
    soft_ref[0] = e / jnp.sum(e, axis=-1, keepdims=True)

    kiota = jax.lax.broadcasted_iota(jnp.int32, logits.shape, 1)
    idx = jnp.min(jnp.where(logits == rowmax, kiota, N_L2), axis=-1,
                  keepdims=True)
    hard_ref[0, 0] = idx.T.astype(jnp.int32)
    b = pl.program_id(0)
    gidx_ref[0, 0] = (idx.T + idx_ref[b] * N_L2).astype(jnp.int32)


def _stage1(local_prosody, codebooks, W1, b1, g1, bt1, l1_indices, temperature):
    grid_spec = pltpu.PrefetchScalarGridSpec(
        num_scalar_prefetch=1,
        grid=(B, NT),
        in_specs=[
            pl.BlockSpec(memory_space=pltpu.SMEM),
            pl.BlockSpec((1, TB, D_MODEL), lambda b, t, i: (b, t, 0)),
            pl.BlockSpec((1, N_L2, BD), lambda b, t, i: (i[b], 0, 0)),
            pl.BlockSpec((D_MODEL, BD), lambda b, t, i: (0, 0)),
            pl.BlockSpec((BD,), lambda b, t, i: (0,)),
            pl.BlockSpec((BD,), lambda b, t, i: (0,)),
            pl.BlockSpec((BD,), lambda b, t, i: (0,)),
        ],
        out_specs=[
            pl.BlockSpec((1, 1, 1, TB), lambda b, t, i: (b, t, 0, 0)),
            pl.BlockSpec((1, TB, N_L2), lambda b, t, i: (b, t, 0)),
            pl.BlockSpec((1, 1, 1, TB), lambda b, t, i: (b, t, 0, 0)),
        ],
        scratch_shapes=[pltpu.VMEM((BD, N_L2), jnp.float32)],
    )
    hard4, soft, gidx4 = pl.pallas_call(
        _stage1_body,
        grid_spec=grid_spec,
        out_shape=[
            jax.ShapeDtypeStruct((B, NT, 1, TB), jnp.int32),
            jax.ShapeDtypeStruct((B, T, N_L2), jnp.float32),
            jax.ShapeDtypeStruct((B, NT, 1, TB), jnp.int32),
        ],
    )(l1_indices.astype(jnp.int32),
      jnp.reshape(jnp.asarray(temperature, jnp.float32), (1,)),
      local_prosody, codebooks, W1, b1, g1, bt1)
    return hard4.reshape(B, T), soft, gidx4.reshape(NW, NCHUNK, CHUNK)


@functools.partial(
    pl.kernel,
    out_type=jax.ShapeDtypeStruct((B * T, BD), jnp.float32),
    mesh=plsc.VectorSubcoreMesh(core_axis_name="c", subcore_axis_name="s"),
    scratch_types=[
        pltpu.VMEM((NCHUNK, CHUNK), jnp.int32),
        pltpu.VMEM((2, CHUNK, BD), jnp.float32),
        pltpu.SemaphoreType.DMA,
        pltpu.SemaphoreType.DMA,
    ],
)
def _sc_gather(table_hbm, gidx_hbm, out_hbm, idx_v, buf_v, sem0, sem1):
    wid = lax.axis_index("c") * 16 + lax.axis_index("s")
    base = wid * RPW

    pltpu.sync_copy(gidx_hbm.at[wid], idx_v)

    sems = (sem0, sem1)
    copies = [None, None]
    for c in range(NCHUNK):
        ph = c % 2
        if copies[ph] is not None:
            copies[ph].wait()
            pltpu.sync_copy(buf_v.at[ph],
                            out_hbm.at[pl.ds(base + (c - 2) * CHUNK, CHUNK)])
        copies[ph] = pltpu.async_copy(table_hbm.at[idx_v.at[c]],
                                      buf_v.at[ph], sems[ph])
    for c in range(NCHUNK - 2, NCHUNK):
        ph = c % 2
        copies[ph].wait()
        pltpu.sync_copy(buf_v.at[ph],
                        out_hbm.at[pl.ds(base + c * CHUNK, CHUNK)])


def _stage3_body(e_ref, W2_ref, b2_ref, g2_ref, bt2_ref, out_ref):
    e0 = jnp.dot(e_ref[...], W2_ref[...],
                 preferred_element_type=jnp.float32) + b2_ref[...]
    out_ref[...] = _ln(e0, g2_ref[...], bt2_ref[...])


def _stage3(emb_low, W2, b2, g2, bt2):
    return pl.pallas_call(
        _stage3_body,
        grid=(B * T // TB3,),
        in_specs=[
            pl.BlockSpec((TB3, BD), lambda t: (t, 0)),
            pl.BlockSpec((BD, D_MODEL), lambda t: (0, 0)),
            pl.BlockSpec((D_MODEL,), lambda t: (0,)),
            pl.BlockSpec((D_MODEL,), lambda t: (0,)),
            pl.BlockSpec((D_MODEL,), lambda t: (0,)),
        ],
        out_specs=pl.BlockSpec((TB3, D_MODEL), lambda t: (t, 0)),
        out_shape=jax.ShapeDtypeStruct((B * T, D_MODEL), jnp.float32),
    )(emb_low, W2, b2, g2, bt2)


@jax.jit
def _run(local_prosody, codebooks, W1, b1, g1, bt1, W2, b2, g2, bt2,
         l1_indices, temperature):
    hard, soft, gidx = _stage1(local_prosody, codebooks, W1, b1, g1, bt1,
                               l1_indices, temperature)
    table = codebooks.reshape(N_L1 * N_L2, BD)
    emb_low = _sc_gather(table, gidx)
    emb = _stage3(emb_low, W2, b2, g2, bt2)
    return (hard, soft, emb.reshape(B, T, D_MODEL),
            emb_low.reshape(B, T, BD))


def kernel(local_prosody, codebooks, W1, b1, g1, bt1, W2, b2, g2, bt2,
           l1_indices, temperature):
    return _run(local_prosody, codebooks, W1, b1, g1, bt1, W2, b2, g2, bt2,
                l1_indices, temperature)


# speedup vs baseline: 1.0660x; 1.0660x over previous
import functools

import jax
import jax.numpy as jnp
from jax import lax
from jax.experimental import pallas as pl
from jax.experimental.pallas import tpu as pltpu
from jax.experimental.pallas import tpu_sc as plsc

B, T = 8, 2048
D_MODEL = 1024
N_L1 = 8
N_L2 = 1024
BD = 256
TB = 512
NT = T // TB

NW = 32
RPW = (B * T) // NW
CHUNK = 128
NCHUNK = RPW // CHUNK
WPB = T // RPW

TB3 = 2048


def _ln(x, g, b, eps=1e-5):
    m = jnp.mean(x, axis=-1, keepdims=True)
    v = jnp.mean((x - m) ** 2, axis=-1, keepdims=True)
    return (x - m) / jnp.sqrt(v + eps) * g + b


def _stage1_body(idx_ref, temp_ref, x_ref, cb_ref, W1_ref, b1_ref, g1_ref,
                 bt1_ref, hard_ref, soft_ref, gidx_ref):
    x = x_ref[0]
    temp = temp_ref[0]

    cb = cb_ref[0]
    cb_inv = 1.0 / jnp.maximum(
        jnp.sqrt(jnp.sum(cb * cb, axis=-1, keepdims=True)), 1e-12)
    cbn = cb * cb_inv

    h0 = jnp.dot(x, W1_ref[...], preferred_element_type=jnp.float32) + b1_ref[...]
    h = _ln(h0, g1_ref[...], bt1_ref[...])
    hn = h / jnp.maximum(jnp.sqrt(jnp.sum(h * h, axis=-1, keepdims=True)), 1e-12)

    logits = jnp.dot(hn, cbn.T, preferred_element_type=jnp.float32) / temp

    rowmax = jnp.max(logits, axis=-1, keepdims=True)
    e = jnp.exp(logits - rowmax)
    soft_ref[0] = e / jnp.sum(e, axis=-1, keepdims=True)

    kiota = jax.lax.broadcasted_iota(jnp.int32, logits.shape, 1)
    idx = jnp.min(jnp.where(logits == rowmax, kiota, N_L2), axis=-1,
                  keepdims=True)
    hard_ref[0, 0] = idx.T.astype(jnp.int32)
    b = pl.program_id(0)
    gidx_ref[0, 0] = (idx.T + idx_ref[b] * N_L2).astype(jnp.int32)


def _stage1(local_prosody, codebooks, W1, b1, g1, bt1, l1_indices, temperature):
    grid_spec = pltpu.PrefetchScalarGridSpec(
        num_scalar_prefetch=1,
        grid=(B, NT),
        in_specs=[
            pl.BlockSpec(memory_space=pltpu.SMEM),
            pl.BlockSpec((1, TB, D_MODEL), lambda b, t, i: (b, t, 0)),
            pl.BlockSpec((1, N_L2, BD), lambda b, t, i: (i[b], 0, 0)),
            pl.BlockSpec((D_MODEL, BD), lambda b, t, i: (0, 0)),
            pl.BlockSpec((BD,), lambda b, t, i: (0,)),
            pl.BlockSpec((BD,), lambda b, t, i: (0,)),
            pl.BlockSpec((BD,), lambda b, t, i: (0,)),
        ],
        out_specs=[
            pl.BlockSpec((1, 1, 1, TB), lambda b, t, i: (b, t, 0, 0)),
            pl.BlockSpec((1, TB, N_L2), lambda b, t, i: (b, t, 0)),
            pl.BlockSpec((1, 1, 1, TB), lambda b, t, i: (b, t, 0, 0)),
        ],
    )
    hard4, soft, gidx4 = pl.pallas_call(
        _stage1_body,
        grid_spec=grid_spec,
        out_shape=[
            jax.ShapeDtypeStruct((B, NT, 1, TB), jnp.int32),
            jax.ShapeDtypeStruct((B, T, N_L2), jnp.float32),
            jax.ShapeDtypeStruct((B, NT, 1, TB), jnp.int32),
        ],
    )(l1_indices.astype(jnp.int32),
      jnp.reshape(jnp.asarray(temperature, jnp.float32), (1,)),
      local_prosody, codebooks, W1, b1, g1, bt1)
    return hard4.reshape(B, T), soft, gidx4.reshape(NW, NCHUNK, CHUNK)


@functools.partial(
    pl.kernel,
    out_type=jax.ShapeDtypeStruct((B * T, BD), jnp.float32),
    mesh=plsc.VectorSubcoreMesh(core_axis_name="c", subcore_axis_name="s"),
    scratch_types=[
        pltpu.VMEM((NCHUNK, CHUNK), jnp.int32),
        pltpu.VMEM((3, CHUNK, BD), jnp.float32),
        pltpu.SemaphoreType.DMA,
        pltpu.SemaphoreType.DMA,
        pltpu.SemaphoreType.DMA,
        pltpu.SemaphoreType.DMA,
        pltpu.SemaphoreType.DMA,
        pltpu.SemaphoreType.DMA,
    ],
)
def _sc_gather(table_hbm, gidx_hbm, out_hbm, idx_v, buf_v,
               gs0, gs1, gs2, ws0, ws1, ws2):
    wid = lax.axis_index("c") * 16 + lax.axis_index("s")
    base = wid * RPW

    pltpu.sync_copy(gidx_hbm.at[wid], idx_v)

    gsems = (gs0, gs1, gs2)
    wsems = (ws0, ws1, ws2)
    gathers = [None, None, None]
    writes = [None, None, None]
    for c in range(NCHUNK):
        ph = c % 3
        if writes[ph] is not None:
            writes[ph].wait()
        gathers[ph] = pltpu.async_copy(table_hbm.at[idx_v.at[c]],
                                       buf_v.at[ph], gsems[ph])
        if c >= 1:
            php = (c - 1) % 3
            gathers[php].wait()
            writes[php] = pltpu.async_copy(
                buf_v.at[php],
                out_hbm.at[pl.ds(base + (c - 1) * CHUNK, CHUNK)], wsems[php])
    ph = (NCHUNK - 1) % 3
    gathers[ph].wait()
    writes[ph] = pltpu.async_copy(
        buf_v.at[ph], out_hbm.at[pl.ds(base + (NCHUNK - 1) * CHUNK, CHUNK)],
        wsems[ph])
    for ph in range(3):
        if writes[ph] is not None:
            writes[ph].wait()


def _stage3_body(e_ref, W2_ref, b2_ref, g2_ref, bt2_ref, out_ref):
    e0 = jnp.dot(e_ref[...], W2_ref[...],
                 preferred_element_type=jnp.float32) + b2_ref[...]
    out_ref[...] = _ln(e0, g2_ref[...], bt2_ref[...])


def _stage3(emb_low, W2, b2, g2, bt2):
    return pl.pallas_call(
        _stage3_body,
        grid=(B * T // TB3,),
        in_specs=[
            pl.BlockSpec((TB3, BD), lambda t: (t, 0)),
            pl.BlockSpec((BD, D_MODEL), lambda t: (0, 0)),
            pl.BlockSpec((D_MODEL,), lambda t: (0,)),
            pl.BlockSpec((D_MODEL,), lambda t: (0,)),
            pl.BlockSpec((D_MODEL,), lambda t: (0,)),
        ],
        out_specs=pl.BlockSpec((TB3, D_MODEL), lambda t: (t, 0)),
        out_shape=jax.ShapeDtypeStruct((B * T, D_MODEL), jnp.float32),
    )(emb_low, W2, b2, g2, bt2)


@jax.jit
def _run(local_prosody, codebooks, W1, b1, g1, bt1, W2, b2, g2, bt2,
         l1_indices, temperature):
    hard, soft, gidx = _stage1(local_prosody, codebooks, W1, b1, g1, bt1,
                               l1_indices, temperature)
    table = codebooks.reshape(N_L1 * N_L2, BD)
    emb_low = _sc_gather(table, gidx)
    emb = _stage3(emb_low, W2, b2, g2, bt2)
    return (hard, soft, emb.reshape(B, T, D_MODEL),
            emb_low.reshape(B, T, BD))


def kernel(local_prosody, codebooks, W1, b1, g1, bt1, W2, b2, g2, bt2,
           l1_indices, temperature):
    return _run(local_prosody, codebooks, W1, b1, g1, bt1, W2, b2, g2, bt2,
                l1_indices, temperature)


# speedup vs baseline: 1.0676x; 1.0015x over previous
import functools

import jax
import jax.numpy as jnp
from jax import lax
from jax.experimental import pallas as pl
from jax.experimental.pallas import tpu as pltpu
from jax.experimental.pallas import tpu_sc as plsc

B, T = 8, 2048
D_MODEL = 1024
N_L1 = 8
N_L2 = 1024
BD = 256
TB = 512
NT = T // TB

BH = B // 2
NW = 32
RPW = (BH * T) // NW
CHUNK = 128
NCHUNK = RPW // CHUNK

TB3 = 2048
NT3 = (BH * T) // TB3


def _ln(x, g, b, eps=1e-5):
    m = jnp.mean(x, axis=-1, keepdims=True)
    v = jnp.mean((x - m) ** 2, axis=-1, keepdims=True)
    return (x - m) / jnp.sqrt(v + eps) * g + b


def _make_stage1_body(off):
    def body(*refs):
        (idx_ref, temp_ref, x_ref, cb_ref, W1_ref, b1_ref, g1_ref,
         bt1_ref) = refs[:8]
        hard_ref, soft_ref, gidx_ref = refs[-3:]
        x = x_ref[0]
        cb = cb_ref[0]
        temp = temp_ref[0]

        h0 = (jnp.dot(x, W1_ref[...], preferred_element_type=jnp.float32)
              + b1_ref[...])
        h = _ln(h0, g1_ref[...], bt1_ref[...])
        hn = h / jnp.maximum(
            jnp.sqrt(jnp.sum(h * h, axis=-1, keepdims=True)), 1e-12)

        cb_inv = 1.0 / jnp.maximum(
            jnp.sqrt(jnp.sum(cb * cb, axis=-1, keepdims=True)), 1e-12)
        cbn = cb * cb_inv

        logits = jnp.dot(hn, cbn.T, preferred_element_type=jnp.float32) / temp

        rowmax = jnp.max(logits, axis=-1, keepdims=True)
        e = jnp.exp(logits - rowmax)
        soft_ref[0] = e / jnp.sum(e, axis=-1, keepdims=True)

        kiota = jax.lax.broadcasted_iota(jnp.int32, logits.shape, 1)
        idx = jnp.min(jnp.where(logits == rowmax, kiota, N_L2), axis=-1,
                      keepdims=True)
        hard_ref[0, 0] = idx.T.astype(jnp.int32)
        b = pl.program_id(0)
        gidx_ref[0, 0] = (idx.T + idx_ref[b + off] * N_L2).astype(jnp.int32)

    return body


def _stage1_half(h, prev, local_prosody, codebooks, W1, b1, g1, bt1,
                 l1_indices, temperature):
    off = h * BH

    in_specs = [
        pl.BlockSpec(memory_space=pltpu.SMEM),
        pl.BlockSpec((1, TB, D_MODEL), lambda b, t, i: (b + off, t, 0)),
        pl.BlockSpec((1, N_L2, BD), lambda b, t, i: (i[b + off], 0, 0)),
        pl.BlockSpec((D_MODEL, BD), lambda b, t, i: (0, 0)),
        pl.BlockSpec((BD,), lambda b, t, i: (0,)),
        pl.BlockSpec((BD,), lambda b, t, i: (0,)),
        pl.BlockSpec((BD,), lambda b, t, i: (0,)),
    ]
    args = [l1_indices.astype(jnp.int32),
            jnp.reshape(jnp.asarray(temperature, jnp.float32), (1,)),
            local_prosody, codebooks, W1, b1, g1, bt1]
    aliases = {}
    if prev is not None:
        in_specs += [pl.BlockSpec(memory_space=pl.ANY),
                     pl.BlockSpec(memory_space=pl.ANY)]
        args += [prev[0], prev[1]]
        aliases = {8: 0, 9: 1}

    grid_spec = pltpu.PrefetchScalarGridSpec(
        num_scalar_prefetch=1,
        grid=(BH, NT),
        in_specs=in_specs,
        out_specs=[
            pl.BlockSpec((1, 1, 1, TB), lambda b, t, i: (b + off, t, 0, 0)),
            pl.BlockSpec((1, TB, N_L2), lambda b, t, i: (b + off, t, 0)),
            pl.BlockSpec((1, 1, 1, TB), lambda b, t, i: (b, t, 0, 0)),
        ],
    )
    hard4, soft, gidx4 = pl.pallas_call(
        _make_stage1_body(off),
        grid_spec=grid_spec,
        out_shape=[
            jax.ShapeDtypeStruct((B, NT, 1, TB), jnp.int32),
            jax.ShapeDtypeStruct((B, T, N_L2), jnp.float32),
            jax.ShapeDtypeStruct((BH, NT, 1, TB), jnp.int32),
        ],
        input_output_aliases=aliases,
    )(*args)
    return hard4, soft, gidx4.reshape(NW, NCHUNK, CHUNK)


@functools.partial(
    pl.kernel,
    out_type=jax.ShapeDtypeStruct((BH * T, BD), jnp.float32),
    mesh=plsc.VectorSubcoreMesh(core_axis_name="c", subcore_axis_name="s"),
    scratch_types=[
        pltpu.VMEM((NCHUNK, CHUNK), jnp.int32),
        pltpu.VMEM((2, CHUNK, BD), jnp.float32),
        pltpu.SemaphoreType.DMA,
        pltpu.SemaphoreType.DMA,
        pltpu.SemaphoreType.DMA,
        pltpu.SemaphoreType.DMA,
    ],
)
def _sc_gather(table_hbm, gidx_hbm, out_hbm, idx_v, buf_v, gs0, gs1, ws0, ws1):
    wid = lax.axis_index("c") * 16 + lax.axis_index("s")
    base = wid * RPW

    pltpu.sync_copy(gidx_hbm.at[wid], idx_v)

    gsems = (gs0, gs1)
    wsems = (ws0, ws1)
    gathers = [None, None]
    writes = [None, None]
    for c in range(NCHUNK):
        ph = c % 2
        if writes[ph] is not None:
            writes[ph].wait()
        gathers[ph] = pltpu.async_copy(table_hbm.at[idx_v.at[c]],
                                       buf_v.at[ph], gsems[ph])
        if c >= 1:
            php = (c - 1) % 2
            gathers[php].wait()
            writes[php] = pltpu.async_copy(
                buf_v.at[php],
                out_hbm.at[pl.ds(base + (c - 1) * CHUNK, CHUNK)], wsems[php])
    ph = (NCHUNK - 1) % 2
    gathers[ph].wait()
    writes[ph] = pltpu.async_copy(
        buf_v.at[ph], out_hbm.at[pl.ds(base + (NCHUNK - 1) * CHUNK, CHUNK)],
        wsems[ph])
    for ph in range(2):
        if writes[ph] is not None:
            writes[ph].wait()


def _stage3_body(*refs):
    e_ref, W2_ref, b2_ref, g2_ref, bt2_ref = refs[:5]
    out_ref, elow_ref = refs[-2:]
    e = e_ref[...]
    e0 = jnp.dot(e, W2_ref[...], preferred_element_type=jnp.float32) + b2_ref[...]
    out_ref[...] = _ln(e0, g2_ref[...], bt2_ref[...])
    elow_ref[...] = e


def _stage3_half(h, prev, emb_low_h, W2, b2, g2, bt2):
    off = h * NT3
    in_specs = [
        pl.BlockSpec((TB3, BD), lambda t: (t, 0)),
        pl.BlockSpec((BD, D_MODEL), lambda t: (0, 0)),
        pl.BlockSpec((D_MODEL,), lambda t: (0,)),
        pl.BlockSpec((D_MODEL,), lambda t: (0,)),
        pl.BlockSpec((D_MODEL,), lambda t: (0,)),
    ]
    args = [emb_low_h, W2, b2, g2, bt2]
    aliases = {}
    if prev is not None:
        in_specs += [pl.BlockSpec(memory_space=pl.ANY),
                     pl.BlockSpec(memory_space=pl.ANY)]
        args += [prev[0], prev[1]]
        aliases = {5: 0, 6: 1}
    return pl.pallas_call(
        _stage3_body,
        grid=(NT3,),
        in_specs=in_specs,
        out_specs=[
            pl.BlockSpec((TB3, D_MODEL), lambda t: (t + off, 0)),
            pl.BlockSpec((TB3, BD), lambda t: (t + off, 0)),
        ],
        out_shape=[
            jax.ShapeDtypeStruct((B * T, D_MODEL), jnp.float32),
            jax.ShapeDtypeStruct((B * T, BD), jnp.float32),
        ],
        input_output_aliases=aliases,
    )(*args)


@jax.jit
def _run(local_prosody, codebooks, W1, b1, g1, bt1, W2, b2, g2, bt2,
         l1_indices, temperature):
    table = codebooks.reshape(N_L1 * N_L2, BD)

    hard_a, soft_a, gidx0 = _stage1_half(
        0, None, local_prosody, codebooks, W1, b1, g1, bt1,
        l1_indices, temperature)
    el0 = _sc_gather(table, gidx0)
    hard, soft, gidx1 = _stage1_half(
        1, (hard_a, soft_a), local_prosody, codebooks, W1, b1, g1, bt1,
        l1_indices, temperature)
    el1 = _sc_gather(table, gidx1)
    emb_a, elow_a = _stage3_half(0, None, el0, W2, b2, g2, bt2)
    emb, elow = _stage3_half(1, (emb_a, elow_a), el1, W2, b2, g2, bt2)

    return (hard.reshape(B, T), soft, emb.reshape(B, T, D_MODEL),
            elow.reshape(B, T, BD))


def kernel(local_prosody, codebooks, W1, b1, g1, bt1, W2, b2, g2, bt2,
           l1_indices, temperature):
    return _run(local_prosody, codebooks, W1, b1, g1, bt1, W2, b2, g2, bt2,
                l1_indices, temperature)


# speedup vs baseline: 1.1226x; 1.0514x over previous
import functools

import jax
import jax.numpy as jnp
from jax import lax
from jax.experimental import pallas as pl
from jax.experimental.pallas import tpu as pltpu
from jax.experimental.pallas import tpu_sc as plsc

B, T = 8, 2048
D_MODEL = 1024
N_L1 = 8
N_L2 = 1024
BD = 256
TB = 2048
NT = T // TB

BH = B // 2
NW = 32
RPW = (BH * T) // NW
CHUNK = 128
NCHUNK = RPW // CHUNK

TB3 = 2048
NT3 = (BH * T) // TB3


def _ln(x, g, b, eps=1e-5):
    m = jnp.mean(x, axis=-1, keepdims=True)
    v = jnp.mean((x - m) ** 2, axis=-1, keepdims=True)
    return (x - m) / jnp.sqrt(v + eps) * g + b


def _make_stage1_body(off):
    def body(*refs):
        (idx_ref, temp_ref, x_ref, cb_ref, W1_ref, b1_ref, g1_ref,
         bt1_ref) = refs[:8]
        hard_ref, soft_ref, gidx_ref = refs[-3:]
        x = x_ref[0]
        cb = cb_ref[0]
        temp = temp_ref[0]

        h0 = (jnp.dot(x, W1_ref[...], preferred_element_type=jnp.float32)
              + b1_ref[...])
        h = _ln(h0, g1_ref[...], bt1_ref[...])
        hn = h / jnp.maximum(
            jnp.sqrt(jnp.sum(h * h, axis=-1, keepdims=True)), 1e-12)

        cb_inv = 1.0 / jnp.maximum(
            jnp.sqrt(jnp.sum(cb * cb, axis=-1, keepdims=True)), 1e-12)
        cbn = cb * cb_inv

        logits = jnp.dot(hn, cbn.T, preferred_element_type=jnp.float32) / temp

        rowmax = jnp.max(logits, axis=-1, keepdims=True)
        e = jnp.exp(logits - rowmax)
        soft_ref[0] = e / jnp.sum(e, axis=-1, keepdims=True)

        kiota = jax.lax.broadcasted_iota(jnp.int32, logits.shape, 1)
        idx = jnp.min(jnp.where(logits == rowmax, kiota, N_L2), axis=-1,
                      keepdims=True)
        hard_ref[0, 0] = idx.T.astype(jnp.int32)
        b = pl.program_id(0)
        gidx_ref[0, 0] = (idx.T + idx_ref[b + off] * N_L2).astype(jnp.int32)

    return body


def _stage1_half(h, prev, local_prosody, codebooks, W1, b1, g1, bt1,
                 l1_indices, temperature):
    off = h * BH

    in_specs = [
        pl.BlockSpec(memory_space=pltpu.SMEM),
        pl.BlockSpec((1, TB, D_MODEL), lambda b, t, i: (b + off, t, 0)),
        pl.BlockSpec((1, N_L2, BD), lambda b, t, i: (i[b + off], 0, 0)),
        pl.BlockSpec((D_MODEL, BD), lambda b, t, i: (0, 0)),
        pl.BlockSpec((BD,), lambda b, t, i: (0,)),
        pl.BlockSpec((BD,), lambda b, t, i: (0,)),
        pl.BlockSpec((BD,), lambda b, t, i: (0,)),
    ]
    args = [l1_indices.astype(jnp.int32),
            jnp.reshape(jnp.asarray(temperature, jnp.float32), (1,)),
            local_prosody, codebooks, W1, b1, g1, bt1]
    aliases = {}
    if prev is not None:
        in_specs += [pl.BlockSpec(memory_space=pl.ANY),
                     pl.BlockSpec(memory_space=pl.ANY)]
        args += [prev[0], prev[1]]
        aliases = {8: 0, 9: 1}

    grid_spec = pltpu.PrefetchScalarGridSpec(
        num_scalar_prefetch=1,
        grid=(BH, NT),
        in_specs=in_specs,
        out_specs=[
            pl.BlockSpec((1, 1, 1, TB), lambda b, t, i: (b + off, t, 0, 0)),
            pl.BlockSpec((1, TB, N_L2), lambda b, t, i: (b + off, t, 0)),
            pl.BlockSpec((1, 1, 1, TB), lambda b, t, i: (b, t, 0, 0)),
        ],
    )
    hard4, soft, gidx4 = pl.pallas_call(
        _make_stage1_body(off),
        grid_spec=grid_spec,
        out_shape=[
            jax.ShapeDtypeStruct((B, NT, 1, TB), jnp.int32),
            jax.ShapeDtypeStruct((B, T, N_L2), jnp.float32),
            jax.ShapeDtypeStruct((BH, NT, 1, TB), jnp.int32),
        ],
        input_output_aliases=aliases,
    )(*args)
    return hard4, soft, gidx4.reshape(NW, NCHUNK, CHUNK)


@functools.partial(
    pl.kernel,
    out_type=jax.ShapeDtypeStruct((BH * T, BD), jnp.float32),
    mesh=plsc.VectorSubcoreMesh(core_axis_name="c", subcore_axis_name="s"),
    scratch_types=[
        pltpu.VMEM((NCHUNK, CHUNK), jnp.int32),
        pltpu.VMEM((2, CHUNK, BD), jnp.float32),
        pltpu.SemaphoreType.DMA,
        pltpu.SemaphoreType.DMA,
        pltpu.SemaphoreType.DMA,
        pltpu.SemaphoreType.DMA,
    ],
)
def _sc_gather(table_hbm, gidx_hbm, out_hbm, idx_v, buf_v, gs0, gs1, ws0, ws1):
    wid = lax.axis_index("c") * 16 + lax.axis_index("s")
    base = wid * RPW

    pltpu.sync_copy(gidx_hbm.at[wid], idx_v)

    gsems = (gs0, gs1)
    wsems = (ws0, ws1)
    gathers = [None, None]
    writes = [None, None]
    for c in range(NCHUNK):
        ph = c % 2
        if writes[ph] is not None:
            writes[ph].wait()
        gathers[ph] = pltpu.async_copy(table_hbm.at[idx_v.at[c]],
                                       buf_v.at[ph], gsems[ph])
        if c >= 1:
            php = (c - 1) % 2
            gathers[php].wait()
            writes[php] = pltpu.async_copy(
                buf_v.at[php],
                out_hbm.at[pl.ds(base + (c - 1) * CHUNK, CHUNK)], wsems[php])
    ph = (NCHUNK - 1) % 2
    gathers[ph].wait()
    writes[ph] = pltpu.async_copy(
        buf_v.at[ph], out_hbm.at[pl.ds(base + (NCHUNK - 1) * CHUNK, CHUNK)],
        wsems[ph])
    for ph in range(2):
        if writes[ph] is not None:
            writes[ph].wait()


def _stage3_body(*refs):
    e_ref, W2_ref, b2_ref, g2_ref, bt2_ref = refs[:5]
    out_ref, elow_ref = refs[-2:]
    e = e_ref[...]
    e0 = jnp.dot(e, W2_ref[...], preferred_element_type=jnp.float32) + b2_ref[...]
    out_ref[...] = _ln(e0, g2_ref[...], bt2_ref[...])
    elow_ref[...] = e


def _stage3_half(h, prev, emb_low_h, W2, b2, g2, bt2):
    off = h * NT3
    in_specs = [
        pl.BlockSpec((TB3, BD), lambda t: (t, 0)),
        pl.BlockSpec((BD, D_MODEL), lambda t: (0, 0)),
        pl.BlockSpec((D_MODEL,), lambda t: (0,)),
        pl.BlockSpec((D_MODEL,), lambda t: (0,)),
        pl.BlockSpec((D_MODEL,), lambda t: (0,)),
    ]
    args = [emb_low_h, W2, b2, g2, bt2]
    aliases = {}
    if prev is not None:
        in_specs += [pl.BlockSpec(memory_space=pl.ANY),
                     pl.BlockSpec(memory_space=pl.ANY)]
        args += [prev[0], prev[1]]
        aliases = {5: 0, 6: 1}
    return pl.pallas_call(
        _stage3_body,
        grid=(NT3,),
        in_specs=in_specs,
        out_specs=[
            pl.BlockSpec((TB3, D_MODEL), lambda t: (t + off, 0)),
            pl.BlockSpec((TB3, BD), lambda t: (t + off, 0)),
        ],
        out_shape=[
            jax.ShapeDtypeStruct((B * T, D_MODEL), jnp.float32),
            jax.ShapeDtypeStruct((B * T, BD), jnp.float32),
        ],
        input_output_aliases=aliases,
    )(*args)


@jax.jit
def _run(local_prosody, codebooks, W1, b1, g1, bt1, W2, b2, g2, bt2,
         l1_indices, temperature):
    table = codebooks.reshape(N_L1 * N_L2, BD)

    hard_a, soft_a, gidx0 = _stage1_half(
        0, None, local_prosody, codebooks, W1, b1, g1, bt1,
        l1_indices, temperature)
    el0 = _sc_gather(table, gidx0)
    hard, soft, gidx1 = _stage1_half(
        1, (hard_a, soft_a), local_prosody, codebooks, W1, b1, g1, bt1,
        l1_indices, temperature)
    el1 = _sc_gather(table, gidx1)
    emb_a, elow_a = _stage3_half(0, None, el0, W2, b2, g2, bt2)
    emb, elow = _stage3_half(1, (emb_a, elow_a), el1, W2, b2, g2, bt2)

    return (hard.reshape(B, T), soft, emb.reshape(B, T, D_MODEL),
            elow.reshape(B, T, BD))


def kernel(local_prosody, codebooks, W1, b1, g1, bt1, W2, b2, g2, bt2,
           l1_indices, temperature):
    return _run(local_prosody, codebooks, W1, b1, g1, bt1, W2, b2, g2, bt2,
                l1_indices, temperature)
